# Initial kernel scaffold; baseline (speedup 1.0000x reference)
import jax
import jax.numpy as jnp
from jax.experimental import pallas as pl


def kernel(data_input, adj_indices, adj_values):
    raise NotImplementedError("write your pallas kernel here")


# speedup vs baseline: 31.4987x; 31.4987x over previous
import functools

import jax
import jax.numpy as jnp
from jax import lax
from jax.experimental import pallas as pl
from jax.experimental.pallas import tpu as pltpu
from jax.experimental.pallas import tpu_sc as plsc

N = 4096
L = 16
NC = 2
NS = 16
NW = NC * NS
GCHUNK = 128


def _sc_partials(nb):
    mesh = plsc.VectorSubcoreMesh(core_axis_name="c", subcore_axis_name="s")

    @functools.partial(
        pl.kernel,
        mesh=mesh,
        out_type=jax.ShapeDtypeStruct((NC, N), jnp.float32),
        scratch_types=[
            pltpu.VMEM((nb, GCHUNK), jnp.int32),
            pltpu.VMEM((nb, GCHUNK), jnp.int32),
            pltpu.VMEM((nb, GCHUNK), jnp.float32),
            pltpu.VMEM((nb, GCHUNK), jnp.int32),
            pltpu.VMEM((nb, GCHUNK), jnp.float32),
            pltpu.VMEM((nb, GCHUNK), jnp.float32),
            pltpu.VMEM((N,), jnp.float32),
            pltpu.VMEM_SHARED((N,), jnp.float32),
            pltpu.SemaphoreType.DMA,
        ],
    )
    def sc_kernel(row_hbm, col_hbm, val_hbm, data_hbm, out_hbm,
                  row_v, col_v, val_v, idx_v, gat_v, prod_v,
                  zero_v, acc_sh, sem):
        cid = lax.axis_index("c")
        sid = lax.axis_index("s")
        wid = sid * NC + cid
        pltpu.sync_copy(row_hbm.at[wid], row_v)
        pltpu.sync_copy(col_hbm.at[wid], col_v)
        pltpu.sync_copy(val_hbm.at[wid], val_v)

        def zero_body(i, _):
            zero_v[pl.ds(i * L, L)] = jnp.zeros((L,), jnp.float32)
            return 0

        @pl.when(sid == 0)
        def _():
            lax.fori_loop(0, N // L, zero_body, 0)
            pltpu.sync_copy(zero_v, acc_sh)

        def idx_body(b, _):
            for j in range(GCHUNK // L):
                s = pl.ds(j * L, L)
                idx_v[b, s] = row_v[b, s] * N + col_v[b, s]
            return 0

        lax.fori_loop(0, nb, idx_body, 0)

        def gat_body(b, _):
            pltpu.async_copy(data_hbm.at[idx_v.at[b]], gat_v.at[b], sem).wait()
            for j in range(GCHUNK // L):
                s = pl.ds(j * L, L)
                prod_v[b, s] = val_v[b, s] * gat_v[b, s]
            return 0

        lax.fori_loop(0, nb, gat_body, 0)

        plsc.subcore_barrier()

        def add_body(b, _):
            pltpu.sync_copy(prod_v.at[b], acc_sh.at[row_v.at[b]], add=True)
            return 0

        lax.fori_loop(0, nb, add_body, 0)

        plsc.subcore_barrier()

        @pl.when(sid == 0)
        def _():
            pltpu.sync_copy(acc_sh, out_hbm.at[cid])

    return sc_kernel


def _tc_finish(parts_ref, alpha_ref, araw_ref):
    a = jnp.sum(parts_ref[...], axis=0, keepdims=True)
    araw_ref[...] = a
    e = jnp.exp(a - jnp.max(a))
    alpha_ref[...] = e / jnp.sum(e)


def kernel(data_input, adj_indices, adj_values):
    n = data_input.shape[0]
    nnz = adj_values.shape[0]
    ch = -(-nnz // NW)
    nb = -(-ch // GCHUNK)
    tot = NW * nb * GCHUNK
    pad = tot - nnz

    shp = (NW, nb, GCHUNK)
    row = jnp.pad(adj_indices[0].astype(jnp.int32), (0, pad)).reshape(shp)
    col = jnp.pad(adj_indices[1].astype(jnp.int32), (0, pad)).reshape(shp)
    val = jnp.pad(adj_values, (0, pad)).reshape(shp)
    data_flat = data_input.reshape(-1)

    partials = _sc_partials(nb)(row, col, val, data_flat)

    alpha2, araw2 = pl.pallas_call(
        _tc_finish,
        out_shape=[
            jax.ShapeDtypeStruct((1, n), jnp.float32),
            jax.ShapeDtypeStruct((1, n), jnp.float32),
        ],
    )(partials)
    return alpha2.reshape(n), araw2.reshape(n)


# speedup vs baseline: 43.0340x; 1.3662x over previous
import functools

import jax
import jax.numpy as jnp
from jax import lax
from jax.experimental import pallas as pl
from jax.experimental.pallas import tpu as pltpu
from jax.experimental.pallas import tpu_sc as plsc

N = 4096
L = 16
NC = 2
NS = 16
NW = NC * NS
GCHUNK = 128


def _sc_partials(nb):
    mesh = plsc.VectorSubcoreMesh(core_axis_name="c", subcore_axis_name="s")

    @functools.partial(
        pl.kernel,
        mesh=mesh,
        out_type=jax.ShapeDtypeStruct((NC, N), jnp.float32),
        scratch_types=[
            pltpu.VMEM((nb, GCHUNK), jnp.int32),
            pltpu.VMEM((nb, GCHUNK), jnp.int32),
            pltpu.VMEM((nb, GCHUNK), jnp.float32),
            pltpu.VMEM((nb, GCHUNK), jnp.int32),
            pltpu.VMEM((nb, GCHUNK), jnp.float32),
            pltpu.VMEM((nb, GCHUNK), jnp.float32),
            pltpu.VMEM((N // NS,), jnp.float32),
            pltpu.VMEM_SHARED((N,), jnp.float32),
            pltpu.SemaphoreType.DMA,
            pltpu.SemaphoreType.DMA,
            pltpu.SemaphoreType.DMA,
        ],
    )
    def sc_kernel(row_hbm, col_hbm, val_hbm, data_hbm, out_hbm,
                  row_v, col_v, val_v, idx_v, gat_v, prod_v,
                  zero_v, acc_sh, sem_in, sem_g, sem_a):
        cid = lax.axis_index("c")
        sid = lax.axis_index("s")
        wid = sid * NC + cid
        pltpu.async_copy(row_hbm.at[wid], row_v, sem_in)
        pltpu.async_copy(col_hbm.at[wid], col_v, sem_in)
        pltpu.async_copy(val_hbm.at[wid], val_v, sem_in)

        zslc = N // NS
        def zero_body(i, _):
            zero_v[pl.ds(i * L, L)] = jnp.zeros((L,), jnp.float32)
            return 0

        lax.fori_loop(0, zslc // L, zero_body, 0)
        pltpu.sync_copy(zero_v, acc_sh.at[pl.ds(sid * zslc, zslc)])

        pltpu.make_async_copy(row_hbm.at[wid], row_v, sem_in).wait()
        pltpu.make_async_copy(col_hbm.at[wid], col_v, sem_in).wait()

        def idx_body(b, _):
            for j in range(GCHUNK // L):
                s = pl.ds(j * L, L)
                idx_v[b, s] = row_v[b, s] * N + col_v[b, s]
            pltpu.async_copy(data_hbm.at[idx_v.at[b]], gat_v.at[b], sem_g)
            return 0

        lax.fori_loop(0, nb, idx_body, 0)

        pltpu.make_async_copy(val_hbm.at[wid], val_v, sem_in).wait()
        plsc.subcore_barrier()

        def prod_body(b, _):
            pltpu.make_async_copy(data_hbm.at[idx_v.at[b]], gat_v.at[b],
                                  sem_g).wait()
            for j in range(GCHUNK // L):
                s = pl.ds(j * L, L)
                prod_v[b, s] = val_v[b, s] * gat_v[b, s]
            pltpu.async_copy(prod_v.at[b], acc_sh.at[row_v.at[b]], sem_a,
                             add=True)
            return 0

        lax.fori_loop(0, nb, prod_body, 0)

        def drain_body(b, _):
            pltpu.make_async_copy(prod_v.at[b], acc_sh.at[row_v.at[b]],
                                  sem_a).wait()
            return 0

        lax.fori_loop(0, nb, drain_body, 0)

        plsc.subcore_barrier()

        @pl.when(sid == 0)
        def _():
            pltpu.sync_copy(acc_sh, out_hbm.at[cid])

    return sc_kernel


def _tc_finish(parts_ref, alpha_ref, araw_ref):
    a = jnp.sum(parts_ref[...], axis=0, keepdims=True)
    araw_ref[...] = a
    e = jnp.exp(a - jnp.max(a))
    alpha_ref[...] = e / jnp.sum(e)


def kernel(data_input, adj_indices, adj_values):
    n = data_input.shape[0]
    nnz = adj_values.shape[0]
    ch = -(-nnz // NW)
    nb = -(-ch // GCHUNK)
    tot = NW * nb * GCHUNK
    pad = tot - nnz

    shp = (NW, nb, GCHUNK)
    row = jnp.pad(adj_indices[0].astype(jnp.int32), (0, pad)).reshape(shp)
    col = jnp.pad(adj_indices[1].astype(jnp.int32), (0, pad)).reshape(shp)
    val = jnp.pad(adj_values, (0, pad)).reshape(shp)
    data_flat = data_input.reshape(-1)

    partials = _sc_partials(nb)(row, col, val, data_flat)

    alpha2, araw2 = pl.pallas_call(
        _tc_finish,
        out_shape=[
            jax.ShapeDtypeStruct((1, n), jnp.float32),
            jax.ShapeDtypeStruct((1, n), jnp.float32),
        ],
    )(partials)
    return alpha2.reshape(n), araw2.reshape(n)


# speedup vs baseline: 43.0582x; 1.0006x over previous
import functools

import jax
import jax.numpy as jnp
from jax import lax
from jax.experimental import pallas as pl
from jax.experimental.pallas import tpu as pltpu
from jax.experimental.pallas import tpu_sc as plsc

N = 4096
L = 16
NC = 2
NS = 16
NW = NC * NS
GCHUNK = 128


def _sc_partials(nb):
    mesh = plsc.VectorSubcoreMesh(core_axis_name="c", subcore_axis_name="s")

    @functools.partial(
        pl.kernel,
        mesh=mesh,
        out_type=jax.ShapeDtypeStruct((NC, N), jnp.float32),
        scratch_types=[
            pltpu.VMEM((nb, GCHUNK), jnp.int32),
            pltpu.VMEM((nb, GCHUNK), jnp.int32),
            pltpu.VMEM((nb, GCHUNK), jnp.float32),
            pltpu.VMEM((nb, GCHUNK), jnp.int32),
            pltpu.VMEM((nb, GCHUNK), jnp.float32),
            pltpu.VMEM((nb, GCHUNK), jnp.float32),
            pltpu.VMEM((N // NS,), jnp.float32),
            pltpu.VMEM_SHARED((N,), jnp.float32),
            pltpu.SemaphoreType.DMA,
            pltpu.SemaphoreType.DMA,
            pltpu.SemaphoreType.DMA,
        ],
    )
    def sc_kernel(edges_hbm, data_hbm, out_hbm,
                  row_v, col_v, val_v, idx_v, gat_v, prod_v,
                  zero_v, acc_sh, sem_in, sem_g, sem_a):
        cid = lax.axis_index("c")
        sid = lax.axis_index("s")
        wid = sid * NC + cid
        pltpu.async_copy(edges_hbm.at[0, wid], row_v, sem_in)
        pltpu.async_copy(edges_hbm.at[1, wid], col_v, sem_in)
        pltpu.async_copy(edges_hbm.at[2, wid].bitcast(jnp.float32), val_v, sem_in)

        zslc = N // NS
        def zero_body(i, _):
            zero_v[pl.ds(i * L, L)] = jnp.zeros((L,), jnp.float32)
            return 0

        lax.fori_loop(0, zslc // L, zero_body, 0)
        pltpu.sync_copy(zero_v, acc_sh.at[pl.ds(sid * zslc, zslc)])

        pltpu.make_async_copy(edges_hbm.at[0, wid], row_v, sem_in).wait()
        pltpu.make_async_copy(edges_hbm.at[1, wid], col_v, sem_in).wait()

        def idx_body(b, _):
            for j in range(GCHUNK // L):
                s = pl.ds(j * L, L)
                idx_v[b, s] = row_v[b, s] * N + col_v[b, s]
            pltpu.async_copy(data_hbm.at[idx_v.at[b]], gat_v.at[b], sem_g)
            return 0

        lax.fori_loop(0, nb, idx_body, 0)

        pltpu.make_async_copy(edges_hbm.at[2, wid].bitcast(jnp.float32), val_v,
                              sem_in).wait()
        plsc.subcore_barrier()

        def prod_body(b, _):
            pltpu.make_async_copy(data_hbm.at[idx_v.at[b]], gat_v.at[b],
                                  sem_g).wait()
            for j in range(GCHUNK // L):
                s = pl.ds(j * L, L)
                prod_v[b, s] = val_v[b, s] * gat_v[b, s]
            pltpu.async_copy(prod_v.at[b], acc_sh.at[row_v.at[b]], sem_a,
                             add=True)
            return 0

        lax.fori_loop(0, nb, prod_body, 0)

        def drain_body(b, _):
            pltpu.make_async_copy(prod_v.at[b], acc_sh.at[row_v.at[b]],
                                  sem_a).wait()
            return 0

        lax.fori_loop(0, nb, drain_body, 0)

        plsc.subcore_barrier()

        @pl.when(sid == 0)
        def _():
            pltpu.sync_copy(acc_sh, out_hbm.at[cid])

    return sc_kernel


def _tc_finish(parts_ref, alpha_ref, araw_ref):
    a = jnp.sum(parts_ref[...], axis=0, keepdims=True)
    araw_ref[...] = a
    e = jnp.exp(a - jnp.max(a))
    alpha_ref[...] = e / jnp.sum(e)


def kernel(data_input, adj_indices, adj_values):
    n = data_input.shape[0]
    nnz = adj_values.shape[0]
    ch = -(-nnz // NW)
    nb = -(-ch // GCHUNK)
    tot = NW * nb * GCHUNK
    pad = tot - nnz

    rc = adj_indices.astype(jnp.int32)
    v32 = lax.bitcast_convert_type(adj_values, jnp.int32)[None]
    edges = jnp.pad(jnp.concatenate([rc, v32], axis=0), ((0, 0), (0, pad)))
    edges = edges.reshape(3, NW, nb, GCHUNK)
    data_flat = data_input.reshape(-1)

    partials = _sc_partials(nb)(edges, data_flat)

    alpha2, araw2 = pl.pallas_call(
        _tc_finish,
        out_shape=[
            jax.ShapeDtypeStruct((1, n), jnp.float32),
            jax.ShapeDtypeStruct((1, n), jnp.float32),
        ],
    )(partials)
    return alpha2.reshape(n), araw2.reshape(n)
